# baseline (speedup 1.0000x reference)
import functools

import jax
import jax.numpy as jnp
from jax import lax
from jax.experimental import pallas as pl
from jax.experimental.pallas import tpu as pltpu
from jax.experimental.pallas import tpu_sc as plsc

VOCAB = 1000000
EMBED_DIM = 64
BATCH = 16384
HIST = 50
HIST_PAD = 56

try:
    _info = plsc.get_sparse_core_info()
    _NC, _NS = _info.num_cores, _info.num_subcores
except Exception:
    _NC, _NS = 2, 16

_NW = _NC * _NS
_ROWS_W = BATCH // _NW
_NCHUNK = _ROWS_W
_NBUF = 8
_GLAG = 4
_NGRP = _NCHUNK // _NBUF

assert _ROWS_W * _NW == BATCH
assert _NGRP * _NBUF == _NCHUNK


def _make_gather():
    mesh = plsc.VectorSubcoreMesh(core_axis_name="c", subcore_axis_name="s")

    @functools.partial(
        pl.kernel,
        mesh=mesh,
        out_type=jax.ShapeDtypeStruct((BATCH, HIST, EMBED_DIM), jnp.float32),
        scratch_types=(
            [pltpu.VMEM((_ROWS_W, HIST_PAD), jnp.int32)]
            + [pltpu.VMEM((HIST_PAD, EMBED_DIM), jnp.float32)] * _NBUF
            + [pltpu.SemaphoreType.DMA] * (2 * _NBUF)
        ),
        compiler_params=pltpu.CompilerParams(use_tc_tiling_on_sc=False),
    )
    def gather_kernel(idx_hbm, table_hbm, out_hbm, idx_v, *scratch):
        rows = scratch[:_NBUF]
        gsem = scratch[_NBUF:2 * _NBUF]
        osem = scratch[2 * _NBUF:]

        wid = lax.axis_index("s") * _NC + lax.axis_index("c")
        base = wid * _ROWS_W
        pltpu.sync_copy(idx_hbm.at[pl.ds(base, _ROWS_W)], idx_v)

        def gather_issue(c, b):
            pltpu.async_copy(table_hbm.at[idx_v.at[c]], rows[b], gsem[b])

        def gather_wait(b):
            pltpu.make_async_copy(
                table_hbm.at[idx_v.at[0]], rows[b], gsem[b]).wait()

        def write_issue(c, b):
            pltpu.async_copy(rows[b].at[pl.ds(0, HIST)],
                             out_hbm.at[base + c], osem[b])

        def write_wait(b):
            pltpu.make_async_copy(rows[b].at[pl.ds(0, HIST)],
                                  out_hbm.at[base], osem[b]).wait()

        for s in range(_NBUF):
            gather_issue(s, s)
            if s >= _GLAG:
                gather_wait(s - _GLAG)
                write_issue(s - _GLAG, s - _GLAG)

        @pl.loop(1, _NGRP)
        def _group(t):
            s0 = t * _NBUF
            for j in range(_NBUF):
                write_wait(j)
                gather_issue(s0 + j, j)
                b = (j + _GLAG) % _NBUF
                gather_wait(b)
                write_issue(s0 + j - _GLAG, b)

        for j in range(_GLAG):
            b = (j + _GLAG) % _NBUF
            gather_wait(b)
            write_issue(_NCHUNK - _GLAG + j, b)
        for j in range(_NBUF):
            write_wait(j)

    return gather_kernel


_gather = _make_gather()


def kernel(nids, nfeats):
    nids_p = jnp.pad(nids.astype(jnp.int32), ((0, 0), (0, HIST_PAD - HIST)))
    return _gather(nids_p, nfeats)


# speedup vs baseline: 1.9302x; 1.9302x over previous
import functools

import jax
import jax.numpy as jnp
from jax import lax
from jax.experimental import pallas as pl
from jax.experimental.pallas import tpu as pltpu
from jax.experimental.pallas import tpu_sc as plsc

VOCAB = 1000000
EMBED_DIM = 64
EMBED_PAD = 128
BATCH = 16384
HIST = 50

try:
    _info = plsc.get_sparse_core_info()
    _NC, _NS = _info.num_cores, _info.num_subcores
except Exception:
    _NC, _NS = 2, 16

_NW = _NC * _NS
_BB = 128
_NBLK = BATCH // _BB
_NCHUNK_TOT = HIST * _NBLK
_PER_W = _NCHUNK_TOT // _NW
_NBUF = 4
_NGRP = _PER_W // _NBUF

assert _PER_W * _NW == _NCHUNK_TOT and _NGRP * _NBUF == _PER_W


def _make_gather():
    mesh = plsc.VectorSubcoreMesh(core_axis_name="c", subcore_axis_name="s")

    @functools.partial(
        pl.kernel,
        mesh=mesh,
        out_type=jax.ShapeDtypeStruct((HIST, EMBED_DIM, BATCH), jnp.float32),
        scratch_types=(
            [pltpu.VMEM((_BB,), jnp.int32)] * _NBUF
            + [pltpu.VMEM((_BB, EMBED_PAD), jnp.float32)] * _NBUF
            + [pltpu.VMEM((EMBED_DIM, _BB), jnp.float32)] * _NBUF
            + [pltpu.SemaphoreType.DMA] * (3 * _NBUF)
        ),
        compiler_params=pltpu.CompilerParams(
            use_tc_tiling_on_sc=True, needs_layout_passes=False),
    )
    def gather_kernel(nT_hbm, tab_hbm, out_hbm, *scratch):
        offs = scratch[:_NBUF]
        rows = scratch[_NBUF:2 * _NBUF]
        tbuf = scratch[2 * _NBUF:3 * _NBUF]
        fsem = scratch[3 * _NBUF:4 * _NBUF]
        gsem = scratch[4 * _NBUF:5 * _NBUF]
        osem = scratch[5 * _NBUF:]

        wid = lax.axis_index("s") * _NC + lax.axis_index("c")
        cbase = wid * _PER_W

        def offs_issue(c, b):
            ch = cbase + c
            h = ch // _NBLK
            b0 = (ch % _NBLK) * _BB
            pltpu.async_copy(nT_hbm.at[h, pl.ds(b0, _BB)], offs[b], fsem[b])

        def offs_wait(b):
            pltpu.make_async_copy(
                nT_hbm.at[0, pl.ds(0, _BB)], offs[b], fsem[b]).wait()

        def gather_issue(b):
            pltpu.async_copy(tab_hbm.at[offs[b]], rows[b], gsem[b])

        def gather_wait(b):
            pltpu.make_async_copy(
                tab_hbm.at[offs[b]], rows[b], gsem[b]).wait()

        def transpose(b):
            @pl.loop(0, _BB // 16)
            def _t(kk):
                row0 = kk * 16
                ridx = lax.iota(jnp.int32, 16) + row0
                for d in range(EMBED_DIM):
                    cidx = jnp.full((16,), d, jnp.int32)
                    vec = plsc.load_gather(rows[b], [ridx, cidx])
                    tbuf[b][d, pl.ds(row0, 16)] = vec

        def write_issue(c, b):
            ch = cbase + c
            h = ch // _NBLK
            b0 = (ch % _NBLK) * _BB
            pltpu.async_copy(
                tbuf[b], out_hbm.at[h, :, pl.ds(b0, _BB)], osem[b])

        def write_wait(b):
            pltpu.make_async_copy(
                tbuf[b], out_hbm.at[0, :, pl.ds(0, _BB)], osem[b]).wait()

        def iter_body(c, j, do_osem_wait, do_offs, do_gather):
            gather_wait(j)
            if do_osem_wait:
                write_wait(j)
            transpose(j)
            write_issue(c, j)
            if do_offs:
                offs_issue(c + _NBUF, j)
            if do_gather:
                b2 = (j + 2) % _NBUF
                offs_wait(b2)
                gather_issue(b2)

        for b in range(_NBUF):
            offs_issue(b, b)
        for b in range(2):
            offs_wait(b)
            gather_issue(b)

        for j in range(_NBUF):
            iter_body(j, j, False, True, True)

        @pl.loop(1, _NGRP - 1)
        def _group(t):
            c0 = t * _NBUF
            for j in range(_NBUF):
                iter_body(c0 + j, j, True, True, True)

        c0 = (_NGRP - 1) * _NBUF
        for j in range(_NBUF):
            iter_body(c0 + j, j, True, False, j < 2)

        for j in range(_NBUF):
            write_wait(j)

    return gather_kernel


_gather = _make_gather()


def kernel(nids, nfeats):
    nT = jnp.swapaxes(nids.astype(jnp.int32), 0, 1)
    tabP = jnp.pad(nfeats, ((0, 0), (0, EMBED_PAD - EMBED_DIM)))
    oT = _gather(nT, tabP)
    return jnp.transpose(oT, (2, 0, 1))


# speedup vs baseline: 2.7266x; 1.4126x over previous
import functools

import jax
import jax.numpy as jnp
from jax import lax
from jax.experimental import pallas as pl
from jax.experimental.pallas import tpu as pltpu
from jax.experimental.pallas import tpu_sc as plsc

VOCAB = 1000000
EMBED_DIM = 64
BATCH = 16384
HIST = 50

try:
    _info = plsc.get_sparse_core_info()
    _NC, _NS = _info.num_cores, _info.num_subcores
except Exception:
    _NC, _NS = 2, 16

_NW = _NC * _NS
_B = BATCH * HIST
_CHUNK = 128
_PER_W = _B // _NW
_NCHUNK = _PER_W // _CHUNK
_NBUF = 8
_GLAG = 4
_NGRP = _NCHUNK // _NBUF

assert _PER_W * _NW == _B and _NGRP * _NBUF == _NCHUNK


def _make_gather():
    mesh = plsc.VectorSubcoreMesh(core_axis_name="c", subcore_axis_name="s")

    @functools.partial(
        pl.kernel,
        mesh=mesh,
        out_type=jax.ShapeDtypeStruct((_B, EMBED_DIM), jnp.float32),
        scratch_types=(
            [pltpu.VMEM((_NCHUNK, _CHUNK), jnp.int32)]
            + [pltpu.VMEM((_CHUNK, EMBED_DIM), jnp.float32)] * _NBUF
            + [pltpu.SemaphoreType.DMA] * (2 * _NBUF)
        ),
        compiler_params=pltpu.CompilerParams(use_tc_tiling_on_sc=False),
    )
    def gather_kernel(idx_hbm, table_hbm, out_hbm, idx_v, *scratch):
        rows = scratch[:_NBUF]
        gsem = scratch[_NBUF:2 * _NBUF]
        osem = scratch[2 * _NBUF:]

        wid = lax.axis_index("s") * _NC + lax.axis_index("c")
        base = wid * _PER_W
        pltpu.sync_copy(idx_hbm.at[wid], idx_v)

        def gather_issue(c, b):
            pltpu.async_copy(table_hbm.at[idx_v.at[c]], rows[b], gsem[b])

        def gather_wait(b):
            pltpu.make_async_copy(
                table_hbm.at[idx_v.at[0]], rows[b], gsem[b]).wait()

        def write_issue(c, b):
            pltpu.async_copy(
                rows[b], out_hbm.at[pl.ds(base + c * _CHUNK, _CHUNK)], osem[b])

        def write_wait(b):
            pltpu.make_async_copy(
                rows[b], out_hbm.at[pl.ds(base, _CHUNK)], osem[b]).wait()

        for s in range(_NBUF):
            gather_issue(s, s)
            if s >= _GLAG:
                gather_wait(s - _GLAG)
                write_issue(s - _GLAG, s - _GLAG)

        @pl.loop(1, _NGRP)
        def _group(t):
            s0 = t * _NBUF
            for j in range(_NBUF):
                write_wait(j)
                gather_issue(s0 + j, j)
                b = (j + _GLAG) % _NBUF
                gather_wait(b)
                write_issue(s0 + j - _GLAG, b)

        for j in range(_GLAG):
            b = (j + _GLAG) % _NBUF
            gather_wait(b)
            write_issue(_NCHUNK - _GLAG + j, b)
        for j in range(_NBUF):
            write_wait(j)

    return gather_kernel


_gather = _make_gather()


def kernel(nids, nfeats):
    idx = nids.reshape(_NW, _NCHUNK, _CHUNK).astype(jnp.int32)
    out = _gather(idx, nfeats)
    return out.reshape(BATCH, HIST, EMBED_DIM)
